# baseline (speedup 1.0000x reference)
import numpy as np
import jax
import jax.numpy as jnp
from jax import lax
from jax.experimental import pallas as pl
from jax.experimental.pallas import tpu as pltpu
from jax.experimental.pallas import tpu_sc as plsc

_B = 2
_P = 1024
_D = 768
_N = 256
_T_FRAMES = (0, 2, 4, 6)
_S_FRAMES = (0, 1, 2, 3)
_NUM_WORKERS = 32
_TASKS = _B * _N
_TPW = _TASKS // _NUM_WORKERS
_RPW = _TPW * 4
_LANE = 16
_CHUNKS = _D // _LANE
_SCALE = 1.0 / (3.0 * _B * _N * _D)
_GROUPS = 4
_TPG = _TPW // _GROUPS
_RPG = _TPG * 4
_NBUF = 3

_REF_PERM = np.array([694, 690, 379, 696, 476, 227, 210, 79, 71, 593, 406, 96, 590, 984, 596, 569, 133, 783, 627, 931, 665, 556, 961, 212, 816, 17, 740, 910, 27, 440, 430, 529, 185, 42, 300, 558, 868, 344, 481, 462, 275, 108, 294, 188, 302, 637, 574, 538, 468, 680, 771, 625, 653, 211, 495, 615, 859, 720, 754, 908, 274, 391, 78, 433, 714, 760, 999, 801, 681, 32, 519, 689, 594, 455, 489, 307, 578, 628, 716, 403, 312, 545, 1, 866, 152, 856, 423, 948, 296, 333, 995, 726, 1019, 911, 1009, 80, 553, 583, 969, 528, 393, 190, 709, 707, 83, 799, 925, 678, 687, 838, 959, 310, 946, 303, 662, 988, 200, 314, 477, 597, 3, 374, 887, 870, 355, 325, 453, 631, 75, 327, 572, 237, 935, 920, 542, 399, 548, 721, 618, 353, 377, 814, 796, 483, 877, 400, 58, 321, 792, 220, 485, 601, 458, 51, 997, 933, 994, 540, 40, 479, 500, 28, 343, 700, 847, 407, 526, 265, 614, 251, 890, 498, 955, 638, 619, 513, 966, 230, 99, 396, 448, 917, 52, 113, 649, 77, 919, 848, 19, 184, 973, 346, 686, 626, 491, 356, 297, 9, 701, 490, 120, 533, 352, 386, 510, 657, 337, 456, 861, 436, 712, 178, 644, 167, 429, 789, 897, 236, 129, 286, 938, 281, 115, 90, 338, 398, 506, 664, 759, 640, 708, 208, 95, 439, 672, 885, 813, 136, 323, 70, 107, 33, 857, 438, 576, 725, 777, 234, 273, 69, 782, 326, 828, 375, 192, 660], dtype=np.int64)
_SHARED_PERM = np.array([83, 1014, 819, 721, 969, 883, 815, 843, 437, 621, 1023, 2, 424, 494, 467, 948, 823, 65, 694, 229, 457, 343, 73, 515, 625, 734, 443, 743, 774, 895, 925, 289, 841, 204, 534, 428, 562, 536, 78, 32, 614, 298, 210, 974, 805, 332, 251, 698, 15, 760, 10, 71, 524, 473, 373, 634, 986, 858, 598, 516, 855, 472, 682, 594, 321, 679, 397, 730, 48, 825, 414, 580, 85, 284, 611, 768, 25, 800, 820, 490, 304, 928, 884, 605, 185, 116, 370, 299, 504, 801, 1016, 208, 471, 830, 136, 499, 656, 451, 813, 109, 114, 453, 243, 912, 252, 954, 657, 170, 640, 264, 407, 962, 521, 438, 175, 809, 692, 896, 140, 590, 267, 158, 150, 115, 416, 607, 1001, 636, 603, 129, 255, 817, 835, 804, 461, 648, 664, 567, 996, 345, 949, 247, 188, 838, 377, 329, 864, 399, 444, 77, 28, 599, 1018, 290, 944, 157, 860, 674, 159, 106, 93, 877, 816, 899, 271, 92, 0, 82, 994, 992, 346, 49, 385, 1013, 173, 477, 922, 609, 865, 69, 849, 227, 187, 1000, 266, 918, 151, 87, 132, 89, 104, 793, 866, 885, 478, 75, 990, 699, 411, 4, 90, 166, 583, 707, 882, 60, 966, 288, 857, 450, 981, 552, 84, 42, 295, 147, 531, 695, 550, 744, 21, 194, 790, 248, 776, 977, 852, 557, 128, 870, 160, 881, 112, 205, 72, 448, 938, 797, 226, 287, 256, 320, 427, 957, 953, 919, 11, 799, 174, 363, 20, 401, 659, 74, 541, 1019], dtype=np.int64)


def _row_indices():
    ref_perm = _REF_PERM
    shared_perm = _SHARED_PERM
    t_idx = np.empty((_B, _N, 4), dtype=np.int64)
    s_idx = np.empty((_B, _N, 4), dtype=np.int64)
    for b in range(_B):
        for q in range(4):
            perm = ref_perm if q == 0 else shared_perm
            t_idx[b, :, q] = (b * 8 + _T_FRAMES[q]) * _P + perm
            s_idx[b, :, q] = (b * 4 + _S_FRAMES[q]) * _P + perm
    t_g = t_idx.reshape(_NUM_WORKERS, _GROUPS, _RPG)
    s_g = s_idx.reshape(_NUM_WORKERS, _GROUPS, _RPG)
    return np.concatenate([t_g, s_g], axis=1).astype(np.int32)


_IDX_NP = _row_indices()


def _huber(d):
    a = jnp.abs(d)
    return jnp.where(a < 0.5, a * a, a - 0.25)


def _sc_body(t2d, s2d, idx, out, idx_v, trows, srows, accv, *sems):
    nc = plsc.get_sparse_core_info().num_cores
    wid = lax.axis_index("s") * nc + lax.axis_index("c")
    pltpu.sync_copy(idx.at[wid], idx_v)

    copies = {}

    def start(g):
        s = g % _NBUF
        copies[g] = (
            pltpu.async_copy(t2d.at[idx_v.at[g]], trows.at[s], sems[2 * s]),
            pltpu.async_copy(s2d.at[idx_v.at[_GROUPS + g]], srows.at[s],
                             sems[2 * s + 1]),
        )

    def compute(s, acc):
        def task_body(k, carry):
            def chunk_body(j, carry2):
                ar, ao = carry2
                c = j * _LANE
                r0 = 4 * k
                tr = trows[s, r0, pl.ds(c, _LANE)]
                sr = srows[s, r0, pl.ds(c, _LANE)]
                dr = sr - tr
                ar = ar + _huber(dr)
                for p in range(1, 4):
                    tp = trows[s, r0 + p, pl.ds(c, _LANE)]
                    sp = srows[s, r0 + p, pl.ds(c, _LANE)]
                    dp = sp - tp
                    ao = ao + _huber(dp) + _huber(dr - dp)
                return (ar, ao)
            return lax.fori_loop(0, _CHUNKS, chunk_body, carry, unroll=16)
        return lax.fori_loop(0, _TPG, task_body, acc)

    start(0)
    start(1)
    acc = (jnp.zeros((_LANE,), jnp.float32), jnp.zeros((_LANE,), jnp.float32))
    for g in range(_GROUPS):
        ct, cs = copies.pop(g)
        ct.wait()
        cs.wait()
        if g + 2 < _GROUPS:
            start(g + 2)
        acc = compute(g % _NBUF, acc)

    accv[...] = (3.0 * acc[0] + acc[1]) * _SCALE
    pltpu.sync_copy(accv, out.at[wid])


def kernel(teacher_feats, student_feats):
    assert teacher_feats.shape == (_B, 8, _P, _D)
    assert student_feats.shape == (_B, 4, _P, _D)
    t2d = teacher_feats.reshape(_B * 8 * _P, _D)
    s2d = student_feats.reshape(_B * 4 * _P, _D)
    idx = jnp.asarray(_IDX_NP)

    mesh = plsc.VectorSubcoreMesh(core_axis_name="c", subcore_axis_name="s")
    run = pl.kernel(
        _sc_body,
        out_type=jax.ShapeDtypeStruct((_NUM_WORKERS, _LANE), jnp.float32),
        mesh=mesh,
        scratch_types=[
            pltpu.VMEM((2 * _GROUPS, _RPG), jnp.int32),
            pltpu.VMEM((_NBUF, _RPG, _D), jnp.float32),
            pltpu.VMEM((_NBUF, _RPG, _D), jnp.float32),
            pltpu.VMEM((_LANE,), jnp.float32),
        ] + [pltpu.SemaphoreType.DMA] * (2 * _NBUF),
    )
    partials = run(t2d, s2d, idx)
    return jnp.sum(partials)


# speedup vs baseline: 1.7690x; 1.7690x over previous
import numpy as np
import jax
import jax.numpy as jnp
from jax import lax
from jax.experimental import pallas as pl
from jax.experimental.pallas import tpu as pltpu
from jax.experimental.pallas import tpu_sc as plsc

_B = 2
_P = 1024
_D = 768
_N = 256
_T_FRAMES = (0, 2, 4, 6)
_S_FRAMES = (0, 1, 2, 3)
_NUM_WORKERS = 32
_TASKS = _B * _N
_TPW = _TASKS // _NUM_WORKERS
_RPW = _TPW * 4
_LANE = 16
_CHUNKS = _D // _LANE
_SCALE = 1.0 / (3.0 * _B * _N * _D)
_GROUPS = 4
_TPG = _TPW // _GROUPS
_RPG = _TPG * 4
_NBUF = 3

_REF_PERM = np.array([694, 690, 379, 696, 476, 227, 210, 79, 71, 593, 406, 96, 590, 984, 596, 569, 133, 783, 627, 931, 665, 556, 961, 212, 816, 17, 740, 910, 27, 440, 430, 529, 185, 42, 300, 558, 868, 344, 481, 462, 275, 108, 294, 188, 302, 637, 574, 538, 468, 680, 771, 625, 653, 211, 495, 615, 859, 720, 754, 908, 274, 391, 78, 433, 714, 760, 999, 801, 681, 32, 519, 689, 594, 455, 489, 307, 578, 628, 716, 403, 312, 545, 1, 866, 152, 856, 423, 948, 296, 333, 995, 726, 1019, 911, 1009, 80, 553, 583, 969, 528, 393, 190, 709, 707, 83, 799, 925, 678, 687, 838, 959, 310, 946, 303, 662, 988, 200, 314, 477, 597, 3, 374, 887, 870, 355, 325, 453, 631, 75, 327, 572, 237, 935, 920, 542, 399, 548, 721, 618, 353, 377, 814, 796, 483, 877, 400, 58, 321, 792, 220, 485, 601, 458, 51, 997, 933, 994, 540, 40, 479, 500, 28, 343, 700, 847, 407, 526, 265, 614, 251, 890, 498, 955, 638, 619, 513, 966, 230, 99, 396, 448, 917, 52, 113, 649, 77, 919, 848, 19, 184, 973, 346, 686, 626, 491, 356, 297, 9, 701, 490, 120, 533, 352, 386, 510, 657, 337, 456, 861, 436, 712, 178, 644, 167, 429, 789, 897, 236, 129, 286, 938, 281, 115, 90, 338, 398, 506, 664, 759, 640, 708, 208, 95, 439, 672, 885, 813, 136, 323, 70, 107, 33, 857, 438, 576, 725, 777, 234, 273, 69, 782, 326, 828, 375, 192, 660], dtype=np.int64)
_SHARED_PERM = np.array([83, 1014, 819, 721, 969, 883, 815, 843, 437, 621, 1023, 2, 424, 494, 467, 948, 823, 65, 694, 229, 457, 343, 73, 515, 625, 734, 443, 743, 774, 895, 925, 289, 841, 204, 534, 428, 562, 536, 78, 32, 614, 298, 210, 974, 805, 332, 251, 698, 15, 760, 10, 71, 524, 473, 373, 634, 986, 858, 598, 516, 855, 472, 682, 594, 321, 679, 397, 730, 48, 825, 414, 580, 85, 284, 611, 768, 25, 800, 820, 490, 304, 928, 884, 605, 185, 116, 370, 299, 504, 801, 1016, 208, 471, 830, 136, 499, 656, 451, 813, 109, 114, 453, 243, 912, 252, 954, 657, 170, 640, 264, 407, 962, 521, 438, 175, 809, 692, 896, 140, 590, 267, 158, 150, 115, 416, 607, 1001, 636, 603, 129, 255, 817, 835, 804, 461, 648, 664, 567, 996, 345, 949, 247, 188, 838, 377, 329, 864, 399, 444, 77, 28, 599, 1018, 290, 944, 157, 860, 674, 159, 106, 93, 877, 816, 899, 271, 92, 0, 82, 994, 992, 346, 49, 385, 1013, 173, 477, 922, 609, 865, 69, 849, 227, 187, 1000, 266, 918, 151, 87, 132, 89, 104, 793, 866, 885, 478, 75, 990, 699, 411, 4, 90, 166, 583, 707, 882, 60, 966, 288, 857, 450, 981, 552, 84, 42, 295, 147, 531, 695, 550, 744, 21, 194, 790, 248, 776, 977, 852, 557, 128, 870, 160, 881, 112, 205, 72, 448, 938, 797, 226, 287, 256, 320, 427, 957, 953, 919, 11, 799, 174, 363, 20, 401, 659, 74, 541, 1019], dtype=np.int64)


def _row_indices():
    ref_perm = _REF_PERM
    shared_perm = _SHARED_PERM
    t_idx = np.empty((_B, _N, 4), dtype=np.int64)
    s_idx = np.empty((_B, _N, 4), dtype=np.int64)
    for b in range(_B):
        for q in range(4):
            perm = ref_perm if q == 0 else shared_perm
            t_idx[b, :, q] = (b * 8 + _T_FRAMES[q]) * _P + perm
            s_idx[b, :, q] = (b * 4 + _S_FRAMES[q]) * _P + perm
    t_g = t_idx.reshape(_NUM_WORKERS, _GROUPS, _RPG)
    s_g = s_idx.reshape(_NUM_WORKERS, _GROUPS, _RPG)
    return np.concatenate([t_g, s_g], axis=1).astype(np.int32)


_IDX_NP = _row_indices()


def _huber(d):
    a = jnp.abs(d)
    return jnp.where(a < 0.5, a * a, a - 0.25)


def _huber16(d):
    a = jnp.abs(d)
    return jnp.where(a < jnp.bfloat16(0.5), a * a, a - jnp.bfloat16(0.25))


def _sc_body(t2d, s2d, idx, out, idx_v, trows, srows, accv, *sems):
    nc = plsc.get_sparse_core_info().num_cores
    wid = lax.axis_index("s") * nc + lax.axis_index("c")
    pltpu.sync_copy(idx.at[wid], idx_v)

    copies = {}

    def start(g):
        s = g % _NBUF
        copies[g] = (
            pltpu.async_copy(t2d.at[idx_v.at[g]], trows.at[s], sems[2 * s]),
            pltpu.async_copy(s2d.at[idx_v.at[_GROUPS + g]], srows.at[s],
                             sems[2 * s + 1]),
        )

    def compute(s, acc):
        def diff16(rows_t, rows_s, r, c):
            lo = rows_s[s, r, pl.ds(c, _LANE)] - rows_t[s, r, pl.ds(c, _LANE)]
            hi = (rows_s[s, r, pl.ds(c + _LANE, _LANE)]
                  - rows_t[s, r, pl.ds(c + _LANE, _LANE)])
            return plsc.pack(lo, hi, format=plsc.PackFormat.INTERLEAVED)

        def task_body(k, carry):
            def chunk_body(j, carry2):
                ar, ao = carry2
                c = j * (2 * _LANE)
                r0 = 4 * k
                drb = diff16(trows, srows, r0, c)
                hr = _huber16(drb)
                ho = None
                for p in range(1, 4):
                    dpb = diff16(trows, srows, r0 + p, c)
                    h = _huber16(dpb) + _huber16(drb - dpb)
                    ho = h if ho is None else ho + h
                hr_lo, hr_hi = plsc.unpack(hr, format=plsc.PackFormat.INTERLEAVED)
                ho_lo, ho_hi = plsc.unpack(ho, format=plsc.PackFormat.INTERLEAVED)
                ar = ar + (hr_lo + hr_hi)
                ao = ao + (ho_lo + ho_hi)
                return (ar, ao)
            return lax.fori_loop(0, _CHUNKS // 2, chunk_body, carry, unroll=4)
        return lax.fori_loop(0, _TPG, task_body, acc)

    start(0)
    start(1)
    acc = (jnp.zeros((_LANE,), jnp.float32), jnp.zeros((_LANE,), jnp.float32))
    for g in range(_GROUPS):
        ct, cs = copies.pop(g)
        ct.wait()
        cs.wait()
        if g + 2 < _GROUPS:
            start(g + 2)
        acc = compute(g % _NBUF, acc)

    accv[...] = (3.0 * acc[0] + acc[1]) * _SCALE
    pltpu.sync_copy(accv, out.at[wid])


def kernel(teacher_feats, student_feats):
    assert teacher_feats.shape == (_B, 8, _P, _D)
    assert student_feats.shape == (_B, 4, _P, _D)
    t2d = teacher_feats.reshape(_B * 8 * _P, _D)
    s2d = student_feats.reshape(_B * 4 * _P, _D)
    idx = jnp.asarray(_IDX_NP)

    mesh = plsc.VectorSubcoreMesh(core_axis_name="c", subcore_axis_name="s")
    run = pl.kernel(
        _sc_body,
        out_type=jax.ShapeDtypeStruct((_NUM_WORKERS, _LANE), jnp.float32),
        mesh=mesh,
        compiler_params=pltpu.CompilerParams(needs_layout_passes=False),
        scratch_types=[
            pltpu.VMEM((2 * _GROUPS, _RPG), jnp.int32),
            pltpu.VMEM((_NBUF, _RPG, _D), jnp.float32),
            pltpu.VMEM((_NBUF, _RPG, _D), jnp.float32),
            pltpu.VMEM((_LANE,), jnp.float32),
        ] + [pltpu.SemaphoreType.DMA] * (2 * _NBUF),
    )
    partials = run(t2d, s2d, idx)
    return jnp.sum(partials)


# speedup vs baseline: 2.6677x; 1.5080x over previous
import numpy as np
import jax
import jax.numpy as jnp
from jax.experimental import pallas as pl
from jax.experimental.pallas import tpu as pltpu

_B = 2
_P = 1024
_D = 768
_N = 256
_SCALE = 1.0 / (3.0 * _B * _N * _D)

_REF_PERM = np.array([694, 690, 379, 696, 476, 227, 210, 79, 71, 593, 406, 96, 590, 984, 596, 569, 133, 783, 627, 931, 665, 556, 961, 212, 816, 17, 740, 910, 27, 440, 430, 529, 185, 42, 300, 558, 868, 344, 481, 462, 275, 108, 294, 188, 302, 637, 574, 538, 468, 680, 771, 625, 653, 211, 495, 615, 859, 720, 754, 908, 274, 391, 78, 433, 714, 760, 999, 801, 681, 32, 519, 689, 594, 455, 489, 307, 578, 628, 716, 403, 312, 545, 1, 866, 152, 856, 423, 948, 296, 333, 995, 726, 1019, 911, 1009, 80, 553, 583, 969, 528, 393, 190, 709, 707, 83, 799, 925, 678, 687, 838, 959, 310, 946, 303, 662, 988, 200, 314, 477, 597, 3, 374, 887, 870, 355, 325, 453, 631, 75, 327, 572, 237, 935, 920, 542, 399, 548, 721, 618, 353, 377, 814, 796, 483, 877, 400, 58, 321, 792, 220, 485, 601, 458, 51, 997, 933, 994, 540, 40, 479, 500, 28, 343, 700, 847, 407, 526, 265, 614, 251, 890, 498, 955, 638, 619, 513, 966, 230, 99, 396, 448, 917, 52, 113, 649, 77, 919, 848, 19, 184, 973, 346, 686, 626, 491, 356, 297, 9, 701, 490, 120, 533, 352, 386, 510, 657, 337, 456, 861, 436, 712, 178, 644, 167, 429, 789, 897, 236, 129, 286, 938, 281, 115, 90, 338, 398, 506, 664, 759, 640, 708, 208, 95, 439, 672, 885, 813, 136, 323, 70, 107, 33, 857, 438, 576, 725, 777, 234, 273, 69, 782, 326, 828, 375, 192, 660], dtype=np.int64)
_SHARED_PERM = np.array([83, 1014, 819, 721, 969, 883, 815, 843, 437, 621, 1023, 2, 424, 494, 467, 948, 823, 65, 694, 229, 457, 343, 73, 515, 625, 734, 443, 743, 774, 895, 925, 289, 841, 204, 534, 428, 562, 536, 78, 32, 614, 298, 210, 974, 805, 332, 251, 698, 15, 760, 10, 71, 524, 473, 373, 634, 986, 858, 598, 516, 855, 472, 682, 594, 321, 679, 397, 730, 48, 825, 414, 580, 85, 284, 611, 768, 25, 800, 820, 490, 304, 928, 884, 605, 185, 116, 370, 299, 504, 801, 1016, 208, 471, 830, 136, 499, 656, 451, 813, 109, 114, 453, 243, 912, 252, 954, 657, 170, 640, 264, 407, 962, 521, 438, 175, 809, 692, 896, 140, 590, 267, 158, 150, 115, 416, 607, 1001, 636, 603, 129, 255, 817, 835, 804, 461, 648, 664, 567, 996, 345, 949, 247, 188, 838, 377, 329, 864, 399, 444, 77, 28, 599, 1018, 290, 944, 157, 860, 674, 159, 106, 93, 877, 816, 899, 271, 92, 0, 82, 994, 992, 346, 49, 385, 1013, 173, 477, 922, 609, 865, 69, 849, 227, 187, 1000, 266, 918, 151, 87, 132, 89, 104, 793, 866, 885, 478, 75, 990, 699, 411, 4, 90, 166, 583, 707, 882, 60, 966, 288, 857, 450, 981, 552, 84, 42, 295, 147, 531, 695, 550, 744, 21, 194, 790, 248, 776, 977, 852, 557, 128, 870, 160, 881, 112, 205, 72, 448, 938, 797, 226, 287, 256, 320, 427, 957, 953, 919, 11, 799, 174, 363, 20, 401, 659, 74, 541, 1019], dtype=np.int64)


def _perm_matrix(perm):
    m = np.zeros((_N, _P), dtype=np.float32)
    m[np.arange(_N), perm] = 1.0
    return m


_PREF_NP = _perm_matrix(_REF_PERM)
_PSH_NP = _perm_matrix(_SHARED_PERM)


def _huber(d):
    a = jnp.abs(d)
    return jnp.where(a < 0.5, a * a, a - 0.25)


def _body(pref_ref, psh_ref, t_ref, s_ref, out_ref, ar_ref, acc_ref):
    i = pl.program_id(0)
    q = jax.lax.rem(i, 4)

    @pl.when(i == 0)
    def _():
        acc_ref[0] = jnp.float32(0.0)

    d = (s_ref[0, 0] - t_ref[0, 0]).astype(jnp.bfloat16)

    @pl.when(q == 0)
    def _():
        ar = jnp.dot(pref_ref[...], d, preferred_element_type=jnp.float32)
        ar_ref[...] = ar
        acc_ref[0] += 3.0 * jnp.sum(_huber(ar))

    @pl.when(q > 0)
    def _():
        ap = jnp.dot(psh_ref[...], d, preferred_element_type=jnp.float32)
        acc_ref[0] += jnp.sum(_huber(ap)) + jnp.sum(_huber(ar_ref[...] - ap))

    @pl.when(i == pl.num_programs(0) - 1)
    def _():
        out_ref[0] = acc_ref[0] * _SCALE


def kernel(teacher_feats, student_feats):
    assert teacher_feats.shape == (_B, 8, _P, _D)
    assert student_feats.shape == (_B, 4, _P, _D)
    pref = jnp.asarray(_PREF_NP, jnp.bfloat16)
    psh = jnp.asarray(_PSH_NP, jnp.bfloat16)

    def pmap(i):
        return (0, 0)

    def tmap(i):
        return (i // 4, 2 * (i % 4), 0, 0)

    def smap(i):
        return (i // 4, i % 4, 0, 0)

    out = pl.pallas_call(
        _body,
        grid=(_B * 4,),
        in_specs=[
            pl.BlockSpec((_N, _P), pmap),
            pl.BlockSpec((_N, _P), pmap),
            pl.BlockSpec((1, 1, _P, _D), tmap),
            pl.BlockSpec((1, 1, _P, _D), smap),
        ],
        out_specs=pl.BlockSpec(memory_space=pltpu.SMEM),
        out_shape=jax.ShapeDtypeStruct((1,), jnp.float32),
        scratch_shapes=[
            pltpu.VMEM((_N, _D), jnp.float32),
            pltpu.SMEM((1,), jnp.float32),
        ],
    )(pref, psh, teacher_feats, student_feats)
    return out[0]
